# Initial kernel scaffold; baseline (speedup 1.0000x reference)
import jax
import jax.numpy as jnp
from jax.experimental import pallas as pl


def kernel(origin_q_emb, origin_d_emb, origin_n_emb, rotation, codebook, ivf_centers, doc_ids, neg_ids, temperature):
    raise NotImplementedError("write your pallas kernel here")


# speedup vs baseline: 2.5472x; 2.5472x over previous
import functools

import jax
import jax.numpy as jnp
from jax import lax
from jax.experimental import pallas as pl
from jax.experimental.pallas import tpu as pltpu
from jax.experimental.pallas import tpu_sc as plsc

B = 1024
NNEG = 2048
D = 768
M = 48
K = 256
DSUB = D // M
N_IVF = 10000
NROWS = B + B + NNEG
NQ = B + NNEG


_NC, _NS = 2, 16
_NW = _NC * _NS
_BPW = NQ // _NW


def _sc_gather(table, ids):
    mesh = plsc.VectorSubcoreMesh(core_axis_name="c", subcore_axis_name="s")

    @functools.partial(
        pl.kernel,
        out_type=jax.ShapeDtypeStruct((NQ, D), jnp.float32),
        mesh=mesh,
        scratch_types=[
            pltpu.VMEM((_BPW,), jnp.int32),
            pltpu.VMEM((_BPW, D), jnp.float32),
            pltpu.SemaphoreType.DMA,
        ],
    )
    def gather_kernel(table_hbm, idx_hbm, out_hbm, idx_v, rows_v, sem):
        wid = lax.axis_index("s") * _NC + lax.axis_index("c")
        base = wid * _BPW
        pltpu.sync_copy(idx_hbm.at[pl.ds(base, _BPW)], idx_v)
        pltpu.async_copy(table_hbm.at[idx_v], rows_v, sem).wait()
        pltpu.sync_copy(rows_v, out_hbm.at[pl.ds(base, _BPW)])

    return gather_kernel(table, ids)


_RBLK = 512


def _rotate_body(x_ref, r_ref, out_ref):
    out_ref[...] = lax.dot_general(
        x_ref[...].astype(jnp.bfloat16), r_ref[...].astype(jnp.bfloat16),
        (((1,), (1,)), ((), ())),
        preferred_element_type=jnp.float32)


def _rotate(x, rotation):
    return pl.pallas_call(
        _rotate_body,
        grid=(NROWS // _RBLK,),
        in_specs=[
            pl.BlockSpec((_RBLK, D), lambda i: (i, 0)),
            pl.BlockSpec((D, D), lambda i: (0, 0)),
        ],
        out_specs=pl.BlockSpec((_RBLK, D), lambda i: (i, 0)),
        out_shape=jax.ShapeDtypeStruct((NROWS, D), jnp.float32),
    )(x, rotation)


_QBLK = 512
_GSUB = 8
_NG = M // _GSUB
_GD = _GSUB * DSUB
_GK = _GSUB * K


def _pq_body(rot_ref, cbd_ref, out_ref):
    rows = rot_ref[...]
    cbd = cbd_ref[0]
    cbd16 = cbd.astype(jnp.bfloat16)
    sm = lax.dot_general(rows.astype(jnp.bfloat16), cbd16,
                         (((1,), (0,)), ((), ())),
                         preferred_element_type=jnp.float32)
    cn = jnp.sum(cbd * cbd, axis=0)
    dist = cn[None, :] - 2.0 * sm
    ohs = []
    for mi in range(_GSUB):
        dsub = dist[:, mi * K:(mi + 1) * K]
        mn = jnp.min(dsub, axis=1, keepdims=True)
        ohs.append((dsub == mn).astype(jnp.bfloat16))
    oh = jnp.concatenate(ohs, axis=1)
    out_ref[...] = lax.dot_general(oh, cbd16, (((1,), (1,)), ((), ())),
                                   preferred_element_type=jnp.float32)


def _pq(rot_dn, cbd):
    return pl.pallas_call(
        _pq_body,
        grid=(NQ // _QBLK, _NG),
        in_specs=[
            pl.BlockSpec((_QBLK, _GD), lambda i, g: (i, g)),
            pl.BlockSpec((1, _GD, _GK), lambda i, g: (g, 0, 0)),
        ],
        out_specs=pl.BlockSpec((_QBLK, _GD), lambda i, g: (i, g)),
        out_shape=jax.ShapeDtypeStruct((NQ, D), jnp.float32),
    )(rot_dn, cbd)


_SRB = 512


def _score_body(a_ref, c_ref, invt_ref, out_ref):
    r = pl.program_id(0)
    a = a_ref[...].astype(jnp.bfloat16)
    c = c_ref[...].astype(jnp.bfloat16)
    s = lax.dot_general(a, c, (((1,), (1,)), ((), ())),
                        preferred_element_type=jnp.float32) * invt_ref[0, 0]
    rowmax = jnp.max(s, axis=1, keepdims=True)
    lse = jnp.log(jnp.sum(jnp.exp(s - rowmax), axis=1, keepdims=True)) + rowmax
    s1 = s[:, :B]
    eye = (lax.broadcasted_iota(jnp.int32, (_SRB, B), 1)
           == lax.broadcasted_iota(jnp.int32, (_SRB, B), 0) + r * _SRB)
    diag = jnp.sum(jnp.where(eye, s1, 0.0), axis=1, keepdims=True)
    partial = -jnp.sum(diag - lse) / B

    @pl.when(r == 0)
    def _():
        out_ref[...] = jnp.full((1, 128), partial, dtype=jnp.float32)

    @pl.when(r > 0)
    def _():
        out_ref[...] = out_ref[...] + partial


def _one_score(a, c, invt):
    out = pl.pallas_call(
        _score_body,
        grid=(B // _SRB,),
        in_specs=[
            pl.BlockSpec((_SRB, D), lambda r: (r, 0)),
            pl.BlockSpec((NQ, D), lambda r: (0, 0)),
            pl.BlockSpec(memory_space=pltpu.SMEM),
        ],
        out_specs=pl.BlockSpec((1, 128), lambda r: (0, 0)),
        out_shape=jax.ShapeDtypeStruct((1, 128), jnp.float32),
    )(a, c, invt)
    return out[0, 0]


def kernel(origin_q_emb, origin_d_emb, origin_n_emb, rotation, codebook,
           ivf_centers, doc_ids, neg_ids, temperature):
    ids = jnp.concatenate([doc_ids.astype(jnp.int32),
                           neg_ids.astype(jnp.int32)])
    gathered = _sc_gather(ivf_centers, ids)
    x = jnp.concatenate([origin_q_emb, origin_d_emb, origin_n_emb], axis=0)
    rot = _rotate(x, rotation)
    rotq = rot[:B]
    cb_r = codebook.reshape(_NG, _GSUB, K, DSUB)
    eye = jnp.eye(_GSUB, dtype=codebook.dtype)
    cbd = jnp.einsum('gmkd,mn->gmdnk', cb_r, eye).reshape(_NG, _GD, _GK)
    quant = _pq(rot[B:], cbd)
    invt = jnp.full((1, 1), 1.0, jnp.float32) / jnp.float32(temperature)
    dense_loss = _one_score(origin_q_emb, x[B:], invt)
    ivf_loss = _one_score(rotq, gathered, invt)
    pq_loss = _one_score(rotq, quant, invt)
    return jnp.stack([dense_loss, ivf_loss, pq_loss])


# speedup vs baseline: 2.9393x; 1.1540x over previous
import functools

import jax
import jax.numpy as jnp
from jax import lax
from jax.experimental import pallas as pl
from jax.experimental.pallas import tpu as pltpu
from jax.experimental.pallas import tpu_sc as plsc

B = 1024
NNEG = 2048
D = 768
M = 48
K = 256
DSUB = D // M
N_IVF = 10000
NROWS = B + B + NNEG
NQ = B + NNEG


_NC, _NS = 2, 16
_NW = _NC * _NS
_BPW = NQ // _NW


def _sc_gather(table, ids):
    mesh = plsc.VectorSubcoreMesh(core_axis_name="c", subcore_axis_name="s")

    @functools.partial(
        pl.kernel,
        out_type=jax.ShapeDtypeStruct((NQ, D), jnp.float32),
        mesh=mesh,
        scratch_types=[
            pltpu.VMEM((_BPW,), jnp.int32),
            pltpu.VMEM((_BPW, D), jnp.float32),
            pltpu.SemaphoreType.DMA,
        ],
    )
    def gather_kernel(table_hbm, idx_hbm, out_hbm, idx_v, rows_v, sem):
        wid = lax.axis_index("s") * _NC + lax.axis_index("c")
        base = wid * _BPW
        pltpu.sync_copy(idx_hbm.at[pl.ds(base, _BPW)], idx_v)
        pltpu.async_copy(table_hbm.at[idx_v], rows_v, sem).wait()
        pltpu.sync_copy(rows_v, out_hbm.at[pl.ds(base, _BPW)])

    return gather_kernel(table, ids)


_RBLK = 512
_QB = B // _RBLK
_DB = B // _RBLK
_NB = NNEG // _RBLK


def _rotate3_body(q_ref, d_ref, n_ref, r_ref, out_ref):
    i = pl.program_id(0)
    r16 = r_ref[...].astype(jnp.bfloat16)

    def rot(src_ref):
        return lax.dot_general(src_ref[...].astype(jnp.bfloat16), r16,
                               (((1,), (1,)), ((), ())),
                               preferred_element_type=jnp.float32)

    @pl.when(i < _QB)
    def _():
        out_ref[...] = rot(q_ref)

    @pl.when(jnp.logical_and(i >= _QB, i < _QB + _DB))
    def _():
        out_ref[...] = rot(d_ref)

    @pl.when(i >= _QB + _DB)
    def _():
        out_ref[...] = rot(n_ref)


def _rotate3(q, d, n, rotation):
    nblk = NROWS // _RBLK
    return pl.pallas_call(
        _rotate3_body,
        grid=(nblk,),
        in_specs=[
            pl.BlockSpec((_RBLK, D), lambda i: (jnp.minimum(i, _QB - 1), 0)),
            pl.BlockSpec((_RBLK, D),
                         lambda i: (jnp.clip(i - _QB, 0, _DB - 1), 0)),
            pl.BlockSpec((_RBLK, D),
                         lambda i: (jnp.clip(i - _QB - _DB, 0, _NB - 1), 0)),
            pl.BlockSpec((D, D), lambda i: (0, 0)),
        ],
        out_specs=pl.BlockSpec((_RBLK, D), lambda i: (i, 0)),
        out_shape=jax.ShapeDtypeStruct((NROWS, D), jnp.float32),
    )(q, d, n, rotation)


_QBLK = 512
_GSUB = 8
_NG = M // _GSUB
_GD = _GSUB * DSUB
_GK = _GSUB * K


def _pq_body(rot_ref, cbd_ref, out_ref, cn_ref):
    i = pl.program_id(1)
    cbd = cbd_ref[0]
    cbd16 = cbd.astype(jnp.bfloat16)

    @pl.when(i == 0)
    def _():
        cn_ref[0:1, :] = jnp.sum(cbd * cbd, axis=0, keepdims=True)

    rows = rot_ref[...]
    sm = lax.dot_general(rows.astype(jnp.bfloat16), cbd16,
                         (((1,), (0,)), ((), ())),
                         preferred_element_type=jnp.float32)
    dist = cn_ref[0:1, :] - 2.0 * sm
    ohs = []
    for mi in range(_GSUB):
        dsub = dist[:, mi * K:(mi + 1) * K]
        mn = jnp.min(dsub, axis=1, keepdims=True)
        ohs.append((dsub == mn).astype(jnp.bfloat16))
    oh = jnp.concatenate(ohs, axis=1)
    out_ref[...] = lax.dot_general(oh, cbd16, (((1,), (1,)), ((), ())),
                                   preferred_element_type=jnp.float32)


def _pq(rot, cbd):
    qoff = B // _QBLK
    return pl.pallas_call(
        _pq_body,
        grid=(_NG, NQ // _QBLK),
        in_specs=[
            pl.BlockSpec((_QBLK, _GD), lambda g, i: (i + qoff, g)),
            pl.BlockSpec((1, _GD, _GK), lambda g, i: (g, 0, 0)),
        ],
        out_specs=pl.BlockSpec((_QBLK, _GD), lambda g, i: (i, g)),
        out_shape=jax.ShapeDtypeStruct((NQ, D), jnp.float32),
        scratch_shapes=[pltpu.VMEM((8, _GK), jnp.float32)],
    )(rot, cbd)


_SRB = 512


def _loss_accum(r, s, invt, out_ref):
    s = s * invt
    rowmax = jnp.max(s, axis=1, keepdims=True)
    lse = jnp.log(jnp.sum(jnp.exp(s - rowmax), axis=1, keepdims=True)) + rowmax
    s1 = s[:, :B]
    eye = (lax.broadcasted_iota(jnp.int32, (_SRB, B), 1)
           == lax.broadcasted_iota(jnp.int32, (_SRB, B), 0) + r * _SRB)
    diag = jnp.sum(jnp.where(eye, s1, 0.0), axis=1, keepdims=True)
    partial = -jnp.sum(diag - lse) / B

    @pl.when(r == 0)
    def _():
        out_ref[...] = jnp.full((1, 128), partial, dtype=jnp.float32)

    @pl.when(r > 0)
    def _():
        out_ref[...] = out_ref[...] + partial


def _score_body(a_ref, c_ref, invt_ref, out_ref):
    r = pl.program_id(0)
    a = a_ref[...].astype(jnp.bfloat16)
    c = c_ref[...].astype(jnp.bfloat16)
    s = lax.dot_general(a, c, (((1,), (1,)), ((), ())),
                        preferred_element_type=jnp.float32)
    _loss_accum(r, s, invt_ref[0, 0], out_ref)


def _dense_body(a_ref, c1_ref, c2_ref, invt_ref, out_ref):
    r = pl.program_id(0)
    a = a_ref[...].astype(jnp.bfloat16)
    s1 = lax.dot_general(a, c1_ref[...].astype(jnp.bfloat16),
                         (((1,), (1,)), ((), ())),
                         preferred_element_type=jnp.float32)
    s2 = lax.dot_general(a, c2_ref[...].astype(jnp.bfloat16),
                         (((1,), (1,)), ((), ())),
                         preferred_element_type=jnp.float32)
    _loss_accum(r, jnp.concatenate([s1, s2], axis=1), invt_ref[0, 0], out_ref)


_OUT_SPEC = dict(
    out_specs=pl.BlockSpec((1, 128), lambda r: (0, 0)),
    out_shape=jax.ShapeDtypeStruct((1, 128), jnp.float32),
)


def _one_score(a, c, invt):
    out = pl.pallas_call(
        _score_body,
        grid=(B // _SRB,),
        in_specs=[
            pl.BlockSpec((_SRB, D), lambda r: (r, 0)),
            pl.BlockSpec((NQ, D), lambda r: (0, 0)),
            pl.BlockSpec(memory_space=pltpu.SMEM),
        ],
        **_OUT_SPEC,
    )(a, c, invt)
    return out[0, 0]


def _dense_score(q, d, n, invt):
    out = pl.pallas_call(
        _dense_body,
        grid=(B // _SRB,),
        in_specs=[
            pl.BlockSpec((_SRB, D), lambda r: (r, 0)),
            pl.BlockSpec((B, D), lambda r: (0, 0)),
            pl.BlockSpec((NNEG, D), lambda r: (0, 0)),
            pl.BlockSpec(memory_space=pltpu.SMEM),
        ],
        **_OUT_SPEC,
    )(q, d, n, invt)
    return out[0, 0]


def kernel(origin_q_emb, origin_d_emb, origin_n_emb, rotation, codebook,
           ivf_centers, doc_ids, neg_ids, temperature):
    ids = jnp.concatenate([doc_ids.astype(jnp.int32),
                           neg_ids.astype(jnp.int32)])
    gathered = _sc_gather(ivf_centers, ids)
    rot = _rotate3(origin_q_emb, origin_d_emb, origin_n_emb, rotation)
    cb_r = codebook.reshape(_NG, _GSUB, K, DSUB)
    eye = jnp.eye(_GSUB, dtype=codebook.dtype)
    cbd = jnp.einsum('gmkd,mn->gmdnk', cb_r, eye).reshape(_NG, _GD, _GK)
    quant = _pq(rot, cbd)
    invt = jnp.full((1, 1), 1.0, jnp.float32) / jnp.float32(temperature)
    dense_loss = _dense_score(origin_q_emb, origin_d_emb, origin_n_emb, invt)
    ivf_loss = _one_score(rot, gathered, invt)
    pq_loss = _one_score(rot, quant, invt)
    return jnp.stack([dense_loss, ivf_loss, pq_loss])


# speedup vs baseline: 3.0001x; 1.0207x over previous
import functools

import jax
import jax.numpy as jnp
from jax import lax
from jax.experimental import pallas as pl
from jax.experimental.pallas import tpu as pltpu
from jax.experimental.pallas import tpu_sc as plsc

B = 1024
NNEG = 2048
D = 768
M = 48
K = 256
DSUB = D // M
N_IVF = 10000
NROWS = B + B + NNEG
NQ = B + NNEG


_NC, _NS = 2, 16
_NW = _NC * _NS
_BPW = NQ // _NW


def _sc_gather(table, ids):
    mesh = plsc.VectorSubcoreMesh(core_axis_name="c", subcore_axis_name="s")

    @functools.partial(
        pl.kernel,
        out_type=jax.ShapeDtypeStruct((NQ, D), jnp.float32),
        mesh=mesh,
        scratch_types=[
            pltpu.VMEM((_BPW,), jnp.int32),
            pltpu.VMEM((_BPW, D), jnp.float32),
            pltpu.SemaphoreType.DMA,
        ],
    )
    def gather_kernel(table_hbm, idx_hbm, out_hbm, idx_v, rows_v, sem):
        wid = lax.axis_index("s") * _NC + lax.axis_index("c")
        base = wid * _BPW
        pltpu.sync_copy(idx_hbm.at[pl.ds(base, _BPW)], idx_v)
        pltpu.async_copy(table_hbm.at[idx_v], rows_v, sem).wait()
        pltpu.sync_copy(rows_v, out_hbm.at[pl.ds(base, _BPW)])

    return gather_kernel(table, ids)


_RBLK = 512
_QB = B // _RBLK
_DB = B // _RBLK
_NB = NNEG // _RBLK


def _rotate3_body(q_ref, d_ref, n_ref, r_ref, out_ref):
    i = pl.program_id(0)
    r16 = r_ref[...].astype(jnp.bfloat16)

    def rot(src_ref):
        return lax.dot_general(src_ref[...].astype(jnp.bfloat16), r16,
                               (((1,), (1,)), ((), ())),
                               preferred_element_type=jnp.float32)

    @pl.when(i < _QB)
    def _():
        out_ref[...] = rot(q_ref)

    @pl.when(jnp.logical_and(i >= _QB, i < _QB + _DB))
    def _():
        out_ref[...] = rot(d_ref)

    @pl.when(i >= _QB + _DB)
    def _():
        out_ref[...] = rot(n_ref)


def _rotate3(q, d, n, rotation):
    nblk = NROWS // _RBLK
    return pl.pallas_call(
        _rotate3_body,
        grid=(nblk,),
        in_specs=[
            pl.BlockSpec((_RBLK, D), lambda i: (jnp.minimum(i, _QB - 1), 0)),
            pl.BlockSpec((_RBLK, D),
                         lambda i: (jnp.clip(i - _QB, 0, _DB - 1), 0)),
            pl.BlockSpec((_RBLK, D),
                         lambda i: (jnp.clip(i - _QB - _DB, 0, _NB - 1), 0)),
            pl.BlockSpec((D, D), lambda i: (0, 0)),
        ],
        out_specs=pl.BlockSpec((_RBLK, D), lambda i: (i, 0)),
        out_shape=jax.ShapeDtypeStruct((NROWS, D), jnp.float32),
    )(q, d, n, rotation)


_QBLK = 512
_GSUB = 8
_NG = M // _GSUB
_GD = _GSUB * DSUB
_GK = _GSUB * K


def _pq_body(rot_ref, cbd_ref, out_ref, cn_ref):
    i = pl.program_id(1)
    cbd = cbd_ref[0]
    cbd16 = cbd.astype(jnp.bfloat16)

    @pl.when(i == 0)
    def _():
        cn_ref[0:1, :] = jnp.sum(cbd * cbd, axis=0, keepdims=True)

    rows = rot_ref[...]
    sm = lax.dot_general(rows.astype(jnp.bfloat16), cbd16,
                         (((1,), (0,)), ((), ())),
                         preferred_element_type=jnp.float32)
    dist = cn_ref[0:1, :] - 2.0 * sm
    ohs = []
    for mi in range(_GSUB):
        dsub = dist[:, mi * K:(mi + 1) * K]
        mn = jnp.min(dsub, axis=1, keepdims=True)
        ohs.append((dsub == mn).astype(jnp.bfloat16))
    oh = jnp.concatenate(ohs, axis=1)
    out_ref[...] = lax.dot_general(oh, cbd16, (((1,), (1,)), ((), ())),
                                   preferred_element_type=jnp.float32)


def _pq(rot, cbd):
    qoff = B // _QBLK
    return pl.pallas_call(
        _pq_body,
        grid=(_NG, NQ // _QBLK),
        in_specs=[
            pl.BlockSpec((_QBLK, _GD), lambda g, i: (i + qoff, g)),
            pl.BlockSpec((1, _GD, _GK), lambda g, i: (g, 0, 0)),
        ],
        out_specs=pl.BlockSpec((_QBLK, _GD), lambda g, i: (i, g)),
        out_shape=jax.ShapeDtypeStruct((NQ, D), jnp.float32),
        scratch_shapes=[pltpu.VMEM((8, _GK), jnp.float32)],
    )(rot, cbd)


_SRB = 512


def _loss_accum(r, s, invt, out_ref):
    s = s * invt
    rowmax = jnp.max(s, axis=1, keepdims=True)
    lse = jnp.log(jnp.sum(jnp.exp(s - rowmax), axis=1, keepdims=True)) + rowmax
    s1 = s[:, :B]
    eye = (lax.broadcasted_iota(jnp.int32, (_SRB, B), 1)
           == lax.broadcasted_iota(jnp.int32, (_SRB, B), 0) + r * _SRB)
    diag = jnp.sum(jnp.where(eye, s1, 0.0), axis=1, keepdims=True)
    partial = -jnp.sum(diag - lse) / B

    @pl.when(r == 0)
    def _():
        out_ref[...] = jnp.full((1, 1, 128), partial, dtype=jnp.float32)

    @pl.when(r > 0)
    def _():
        out_ref[...] = out_ref[...] + partial


def _mm16(a, b):
    return lax.dot_general(a, b.astype(jnp.bfloat16), (((1,), (1,)), ((), ())),
                           preferred_element_type=jnp.float32)


def _losses_body(q_ref, rot_ref, d_ref, n_ref, g_ref, p_ref, invt_ref,
                 out_ref):
    sid = pl.program_id(0)
    r = pl.program_id(1)
    invt = invt_ref[0, 0]

    @pl.when(sid == 0)
    def _():
        a = q_ref[...].astype(jnp.bfloat16)
        s = jnp.concatenate([_mm16(a, d_ref[...]), _mm16(a, n_ref[...])],
                            axis=1)
        _loss_accum(r, s, invt, out_ref)

    @pl.when(sid == 1)
    def _():
        a = rot_ref[...].astype(jnp.bfloat16)
        _loss_accum(r, _mm16(a, g_ref[...]), invt, out_ref)

    @pl.when(sid == 2)
    def _():
        a = rot_ref[...].astype(jnp.bfloat16)
        _loss_accum(r, _mm16(a, p_ref[...]), invt, out_ref)


def _losses(q, rot, d, n, gathered, quant, invt):
    out = pl.pallas_call(
        _losses_body,
        grid=(3, B // _SRB),
        in_specs=[
            pl.BlockSpec((_SRB, D),
                         lambda s, r: (jnp.where(s == 0, r, 1), 0)),
            pl.BlockSpec((_SRB, D),
                         lambda s, r: (jnp.where(s == 0, 0, r), 0)),
            pl.BlockSpec((B, D), lambda s, r: (0, 0)),
            pl.BlockSpec((NNEG, D), lambda s, r: (0, 0)),
            pl.BlockSpec((NQ, D), lambda s, r: (0, 0)),
            pl.BlockSpec((NQ, D), lambda s, r: (0, 0)),
            pl.BlockSpec(memory_space=pltpu.SMEM),
        ],
        out_specs=pl.BlockSpec((1, 1, 128), lambda s, r: (s, 0, 0)),
        out_shape=jax.ShapeDtypeStruct((3, 1, 128), jnp.float32),
    )(q, rot, d, n, gathered, quant, invt)
    return out[:, 0, 0]


def kernel(origin_q_emb, origin_d_emb, origin_n_emb, rotation, codebook,
           ivf_centers, doc_ids, neg_ids, temperature):
    ids = jnp.concatenate([doc_ids.astype(jnp.int32),
                           neg_ids.astype(jnp.int32)])
    gathered = _sc_gather(ivf_centers, ids)
    rot = _rotate3(origin_q_emb, origin_d_emb, origin_n_emb, rotation)
    cb_r = codebook.reshape(_NG, _GSUB, K, DSUB)
    eye = jnp.eye(_GSUB, dtype=codebook.dtype)
    cbd = jnp.einsum('gmkd,mn->gmdnk', cb_r, eye).reshape(_NG, _GD, _GK)
    quant = _pq(rot, cbd)
    invt = jnp.full((1, 1), 1.0, jnp.float32) / jnp.float32(temperature)
    return _losses(origin_q_emb, rot, origin_d_emb, origin_n_emb,
                   gathered, quant, invt)
